# baseline (speedup 1.0000x reference)
import functools

import jax
import jax.numpy as jnp
from jax import lax
from jax.experimental import pallas as pl
from jax.experimental.pallas import tpu as pltpu
from jax.experimental.pallas import tpu_sc as plsc

N = 10000
E = 320000
D_IN = 128
D_HID = 16
D_OUT = 64

NC = 2
NS = 16
NW = NC * NS

CHUNK = 128
CPT = 80
EPT = CPT * CHUNK
E_PAD = EPT * NW
E_ROWS = E_PAD // CHUNK

NACC = 10112
STRIPE = NACC // NS
DUMMY = N

def _deg_body(src_hbm, dst_hbm, zeros_hbm, ones_hbm,
              deg_hbm, dstp_hbm,
              idx_s, idx_d, idx_p, ones_v, acc, sem):
    c = lax.axis_index("c")
    s = lax.axis_index("s")
    tid = c * NS + s
    base_row = tid * CPT
    pltpu.sync_copy(zeros_hbm, acc.at[pl.ds(s * STRIPE, STRIPE)])
    pltpu.sync_copy(ones_hbm, ones_v)
    pltpu.sync_copy(src_hbm.at[pl.ds(base_row, CPT)], idx_s)
    pltpu.sync_copy(dst_hbm.at[pl.ds(base_row, CPT)], idx_d)
    plsc.subcore_barrier()

    def chunk(j, carry):
        for k in range(CHUNK // 16):
            sv = idx_s[j, pl.ds(k * 16, 16)]
            dv = idx_d[j, pl.ds(k * 16, 16)]
            idx_p[j, pl.ds(k * 16, 16)] = jnp.where(sv == dv, jnp.int32(DUMMY), dv)
        pltpu.sync_copy(ones_v, acc.at[idx_p.at[j]], add=True)
        return carry

    lax.fori_loop(0, CPT, chunk, 0)
    pltpu.sync_copy(idx_p, dstp_hbm.at[pl.ds(base_row, CPT)])
    plsc.subcore_barrier()
    pltpu.sync_copy(acc.at[pl.ds(s * STRIPE, STRIPE)],
                    deg_hbm.at[c, pl.ds(s * STRIPE, STRIPE)])


NB = 4


def _prop_body(g_hbm, src_hbm, dstp_hbm, zeros_hbm,
               out_hbm,
               idx_s, idx_d, rows, acc, gs0, gs1, gs2, gs3):
    c = lax.axis_index("c")
    s = lax.axis_index("s")
    tid = c * NS + s
    base_row = tid * CPT
    pltpu.sync_copy(zeros_hbm, acc.at[pl.ds(s * STRIPE, STRIPE)])
    pltpu.sync_copy(src_hbm.at[pl.ds(base_row, CPT)], idx_s.at[pl.ds(0, CPT)])
    pltpu.sync_copy(dstp_hbm.at[pl.ds(base_row, CPT)], idx_d)
    zero16 = jnp.zeros((16,), jnp.int32)
    for r in range(2):
        for k in range(CHUNK // 16):
            idx_s[CPT + r, pl.ds(k * 16, 16)] = zero16
    plsc.subcore_barrier()

    sems = (gs0, gs1, gs2, gs3)

    def gather(j, b):
        return pltpu.async_copy(g_hbm.at[idx_s.at[j]], rows.at[b], sems[b])

    def gather_wait(j, b):
        pltpu.make_async_copy(g_hbm.at[idx_s.at[j]], rows.at[b], sems[b]).wait()

    gather(0, 0)
    gather(1, 1)

    def body(t, carry):
        j = 4 * t
        gather(j + 2, 2)
        gather_wait(j, 0)
        pltpu.sync_copy(rows.at[0], acc.at[idx_d.at[j]], add=True)
        gather(j + 3, 3)
        gather_wait(j + 1, 1)
        pltpu.sync_copy(rows.at[1], acc.at[idx_d.at[j + 1]], add=True)
        gather(j + 4, 0)
        gather_wait(j + 2, 2)
        pltpu.sync_copy(rows.at[2], acc.at[idx_d.at[j + 2]], add=True)
        gather(j + 5, 1)
        gather_wait(j + 3, 3)
        pltpu.sync_copy(rows.at[3], acc.at[idx_d.at[j + 3]], add=True)
        return carry

    lax.fori_loop(0, CPT // 4, body, 0)
    gather_wait(CPT, 0)
    gather_wait(CPT + 1, 1)
    plsc.subcore_barrier()
    pltpu.sync_copy(acc.at[pl.ds(s * STRIPE, STRIPE)],
                    out_hbm.at[c, pl.ds(s * STRIPE, STRIPE)])


@functools.cache
def _sc_kernels():
    mesh = plsc.VectorSubcoreMesh(core_axis_name="c", subcore_axis_name="s",
                                  num_cores=NC, num_subcores=NS)
    params = pltpu.CompilerParams(use_tc_tiling_on_sc=False)
    deg_kernel = pl.kernel(
        _deg_body,
        out_type=(jax.ShapeDtypeStruct((NC, NACC, D_HID), jnp.float32),
                  jax.ShapeDtypeStruct((E_ROWS, CHUNK), jnp.int32)),
        mesh=mesh,
        scratch_types=[
            pltpu.VMEM((CPT, CHUNK), jnp.int32),
            pltpu.VMEM((CPT, CHUNK), jnp.int32),
            pltpu.VMEM((CPT, CHUNK), jnp.int32),
            pltpu.VMEM((CHUNK, D_HID), jnp.float32),
            pltpu.VMEM_SHARED((NACC, D_HID), jnp.float32),
            pltpu.SemaphoreType.DMA,
        ],
        compiler_params=params,
    )
    prop_kernel = pl.kernel(
        _prop_body,
        out_type=jax.ShapeDtypeStruct((NC, NACC, D_HID), jnp.float32),
        mesh=mesh,
        scratch_types=[
            pltpu.VMEM((CPT + 8, CHUNK), jnp.int32),
            pltpu.VMEM((CPT, CHUNK), jnp.int32),
            pltpu.VMEM((NB, CHUNK, D_HID), jnp.float32),
            pltpu.VMEM_SHARED((NACC, D_HID), jnp.float32),
            pltpu.SemaphoreType.DMA,
            pltpu.SemaphoreType.DMA,
            pltpu.SemaphoreType.DMA,
            pltpu.SemaphoreType.DMA,
        ],
        compiler_params=params,
    )
    return deg_kernel, prop_kernel


_RB = 1000


def _lin1_body(deg2_ref, x_ref, w1_ref, dinv_ref, g1_ref):
    deg = deg2_ref[0] + deg2_ref[1] + 1.0
    dinv = lax.rsqrt(deg)
    dinv_ref[...] = dinv
    g1_ref[...] = dinv * jnp.dot(x_ref[...], w1_ref[...],
                                 preferred_element_type=jnp.float32)


def _lin1(deg2, x, W1):
    return pl.pallas_call(
        _lin1_body,
        grid=(N // _RB,),
        in_specs=[
            pl.BlockSpec((NC, _RB, D_HID), lambda i: (0, i, 0)),
            pl.BlockSpec((_RB, D_IN), lambda i: (i, 0)),
            pl.BlockSpec((D_IN, D_HID), lambda i: (0, 0)),
        ],
        out_specs=[
            pl.BlockSpec((_RB, D_HID), lambda i: (i, 0)),
            pl.BlockSpec((_RB, D_HID), lambda i: (i, 0)),
        ],
        out_shape=[jax.ShapeDtypeStruct((N, D_HID), jnp.float32),
                   jax.ShapeDtypeStruct((N, D_HID), jnp.float32)],
    )(deg2, x, W1)


def _hid_body(s1_ref, g1_ref, dinv_ref, b1_ref, g2_ref):
    pre = dinv_ref[...] * (s1_ref[0] + s1_ref[1] + g1_ref[...]) + b1_ref[...]
    g2_ref[...] = dinv_ref[...] * jnp.maximum(pre, 0.0)


def _hid(s1, g1, dinv, b1):
    return pl.pallas_call(
        _hid_body,
        grid=(N // _RB,),
        in_specs=[
            pl.BlockSpec((NC, _RB, D_HID), lambda i: (0, i, 0)),
            pl.BlockSpec((_RB, D_HID), lambda i: (i, 0)),
            pl.BlockSpec((_RB, D_HID), lambda i: (i, 0)),
            pl.BlockSpec((1, D_HID), lambda i: (0, 0)),
        ],
        out_specs=pl.BlockSpec((_RB, D_HID), lambda i: (i, 0)),
        out_shape=jax.ShapeDtypeStruct((N, D_HID), jnp.float32),
    )(s1, g1, dinv, b1)


def _out_body(s2_ref, g2_ref, dinv_ref, w2_ref, b2_ref, out_ref):
    h = dinv_ref[...] * (s2_ref[0] + s2_ref[1] + g2_ref[...])
    z = jnp.dot(h, w2_ref[...], preferred_element_type=jnp.float32) + b2_ref[...]
    m = jnp.max(z, axis=1, keepdims=True)
    lse = jnp.log(jnp.sum(jnp.exp(z - m), axis=1, keepdims=True)) + m
    out_ref[...] = z - lse


def _out(s2, g2, dinv, W2, b2):
    return pl.pallas_call(
        _out_body,
        grid=(N // _RB,),
        in_specs=[
            pl.BlockSpec((NC, _RB, D_HID), lambda i: (0, i, 0)),
            pl.BlockSpec((_RB, D_HID), lambda i: (i, 0)),
            pl.BlockSpec((_RB, D_HID), lambda i: (i, 0)),
            pl.BlockSpec((D_HID, D_OUT), lambda i: (0, 0)),
            pl.BlockSpec((1, D_OUT), lambda i: (0, 0)),
        ],
        out_specs=pl.BlockSpec((_RB, D_OUT), lambda i: (i, 0)),
        out_shape=jax.ShapeDtypeStruct((N, D_OUT), jnp.float32),
    )(s2, g2, dinv, W2, b2)


def kernel(x, edge_index, W1, b1, W2, b2):
    src = edge_index[0].astype(jnp.int32)
    dst = edge_index[1].astype(jnp.int32)
    pad = E_PAD - E
    src_p = jnp.concatenate([src, jnp.zeros((pad,), jnp.int32)]).reshape(E_ROWS, CHUNK)
    dst_p = jnp.concatenate([dst, jnp.full((pad,), DUMMY, jnp.int32)]).reshape(E_ROWS, CHUNK)

    zeros_stripe = jnp.zeros((STRIPE, D_HID), jnp.float32)
    ones_chunk = jnp.ones((CHUNK, D_HID), jnp.float32)

    deg_kernel, prop_kernel = _sc_kernels()
    deg2, dstp = deg_kernel(src_p, dst_p, zeros_stripe, ones_chunk)
    dinv, g1 = _lin1(deg2[:, :N], x, W1)

    s1 = prop_kernel(g1, src_p, dstp, zeros_stripe)
    g2 = _hid(s1[:, :N], g1, dinv, b1.reshape(1, D_HID))

    s2 = prop_kernel(g2, src_p, dstp, zeros_stripe)
    return _out(s2[:, :N], g2, dinv, W2, b2.reshape(1, D_OUT))


# speedup vs baseline: 1.1255x; 1.1255x over previous
import functools

import jax
import jax.numpy as jnp
from jax import lax
from jax.experimental import pallas as pl
from jax.experimental.pallas import tpu as pltpu
from jax.experimental.pallas import tpu_sc as plsc

N = 10000
E = 320000
D_IN = 128
D_HID = 16
D_OUT = 64

NC = 2
NS = 16
NW = NC * NS

CHUNK = 128
CPT = 80
EPT = CPT * CHUNK
E_PAD = EPT * NW
E_ROWS = E_PAD // CHUNK
DROWS = E_ROWS // NS

NACC = 10240
STRIPE = NACC // NS
DUMMY = N

L = 16
NVEC = STRIPE // L


def _take16(y, idx):
    dnums = lax.GatherDimensionNumbers(offset_dims=(), collapsed_slice_dims=(0,),
                                       start_index_map=(0,))
    return lax.gather(y, idx[:, None], dnums, slice_sizes=(1,),
                      mode=lax.GatherScatterMode.PROMISE_IN_BOUNDS)


def _rsqrt16(d):
    i = plsc.bitcast(d, jnp.int32)
    i = jnp.int32(0x5F3759DF) - lax.shift_right_logical(i, 1)
    y = plsc.bitcast(i, jnp.float32)
    for _ in range(3):
        y = y * (1.5 - 0.5 * d * y * y)
    return y


def _stage1_body(u_hbm, src_hbm, dst_hbm, zeros_hbm,
                 s1_hbm, g1_hbm, dinv_hbm, dstp_hbm,
                 hist, sdg, ddg, tmp, dvbuf, g1buf, sps, spd, rows,
                 spm, acc, gs0, gs1):
    c = lax.axis_index("c")
    s = lax.axis_index("s")
    tid = c * NS + s
    zero16 = jnp.zeros((16,), jnp.int32)
    ones16 = jnp.ones((16,), jnp.float32)

    pltpu.sync_copy(zeros_hbm, hist)
    pltpu.sync_copy(src_hbm.at[pl.ds(s * DROWS, DROWS)], sdg)
    pltpu.sync_copy(dst_hbm.at[pl.ds(s * DROWS, DROWS)], ddg)

    def hist_row(j, carry):
        for k in range(CHUNK // L):
            sv = sdg[j, pl.ds(k * L, L)]
            dv = ddg[j, pl.ds(k * L, L)]
            dp = jnp.where(sv == dv, jnp.int32(DUMMY), dv)
            ddg[j, pl.ds(k * L, L)] = dp
            plsc.addupdate_scatter(
                hist, [lax.shift_right_logical(dp, 4), dp & 15], ones16)
        return carry

    lax.fori_loop(0, DROWS, hist_row, 0)

    pltpu.sync_copy(src_hbm.at[pl.ds(tid * CPT, CPT)], sps.at[pl.ds(0, CPT)])
    pltpu.sync_copy(dst_hbm.at[pl.ds(tid * CPT, CPT)], spd)
    slab = c * NACC

    def prep_row(j, carry):
        for k in range(CHUNK // L):
            sv = sps[j, pl.ds(k * L, L)]
            dv = spd[j, pl.ds(k * L, L)]
            spd[j, pl.ds(k * L, L)] = jnp.where(sv == dv, jnp.int32(DUMMY), dv)
            sps[j, pl.ds(k * L, L)] = sv + slab
        return carry

    lax.fori_loop(0, CPT, prep_row, 0)
    for k in range(CHUNK // L):
        sps[CPT, pl.ds(k * L, L)] = zero16 + slab

    pltpu.sync_copy(hist, spm.at[s])
    pltpu.sync_copy(zeros_hbm, acc.at[pl.ds(s * STRIPE, STRIPE)])
    @pl.when(c == 0)
    def _():
        pltpu.sync_copy(ddg, dstp_hbm.at[pl.ds(s * DROWS, DROWS)])
    plsc.subcore_barrier()

    for t in range(NS):
        pltpu.sync_copy(spm.at[t, pl.ds(s * NVEC, NVEC)], tmp.at[t])
    pltpu.sync_copy(u_hbm.at[pl.ds(s * STRIPE, STRIPE)], g1buf)

    lane_ids = [jnp.full((L,), i, jnp.int32) for i in range(L)]

    def dinv_vec(v, carry):
        deg = tmp[0, v, :] + 1.0
        for t in range(1, NS):
            deg = deg + tmp[t, v, :]
        y = _rsqrt16(deg)
        dvbuf[pl.ds(v * L, L)] = y
        for i in range(L):
            row = v * L + i
            g1buf[row, :] = g1buf[row, :] * _take16(y, lane_ids[i])
        return carry

    lax.fori_loop(0, NVEC, dinv_vec, 0)
    pltpu.sync_copy(g1buf, g1_hbm.at[pl.ds(slab + s * STRIPE, STRIPE)])
    pltpu.sync_copy(dvbuf, dinv_hbm.at[c, pl.ds(s * STRIPE, STRIPE)])
    plsc.subcore_barrier()

    sems = (gs0, gs1)

    def gather(j, b):
        return pltpu.async_copy(g1_hbm.at[sps.at[j]], rows.at[b], sems[b])

    def gather_wait(j, b):
        pltpu.make_async_copy(g1_hbm.at[sps.at[j]], rows.at[b], sems[b]).wait()

    gather(0, 0)

    def prop(t, carry):
        j = 2 * t
        gather(j + 1, 1)
        gather_wait(j, 0)
        pltpu.sync_copy(rows.at[0], acc.at[spd.at[j]], add=True)
        gather(j + 2, 0)
        gather_wait(j + 1, 1)
        pltpu.sync_copy(rows.at[1], acc.at[spd.at[j + 1]], add=True)
        return carry

    lax.fori_loop(0, CPT // 2, prop, 0)
    gather_wait(CPT, 0)
    plsc.subcore_barrier()
    pltpu.sync_copy(acc.at[pl.ds(s * STRIPE, STRIPE)],
                    s1_hbm.at[c, pl.ds(s * STRIPE, STRIPE)])


def _prop_body(g_hbm, src_hbm, dstp_hbm, zeros_hbm,
               out_hbm,
               idx_s, idx_d, rows, acc, gs0, gs1):
    c = lax.axis_index("c")
    s = lax.axis_index("s")
    tid = c * NS + s
    base_row = tid * CPT
    pltpu.sync_copy(zeros_hbm, acc.at[pl.ds(s * STRIPE, STRIPE)])
    pltpu.sync_copy(src_hbm.at[pl.ds(base_row, CPT)], idx_s.at[pl.ds(0, CPT)])
    pltpu.sync_copy(dstp_hbm.at[pl.ds(base_row, CPT)], idx_d)
    zero16 = jnp.zeros((16,), jnp.int32)
    for k in range(CHUNK // L):
        idx_s[CPT, pl.ds(k * L, L)] = zero16
    plsc.subcore_barrier()

    sems = (gs0, gs1)

    def gather(j, b):
        return pltpu.async_copy(g_hbm.at[idx_s.at[j]], rows.at[b], sems[b])

    def gather_wait(j, b):
        pltpu.make_async_copy(g_hbm.at[idx_s.at[j]], rows.at[b], sems[b]).wait()

    gather(0, 0)

    def body(t, carry):
        j = 2 * t
        gather(j + 1, 1)
        gather_wait(j, 0)
        pltpu.sync_copy(rows.at[0], acc.at[idx_d.at[j]], add=True)
        gather(j + 2, 0)
        gather_wait(j + 1, 1)
        pltpu.sync_copy(rows.at[1], acc.at[idx_d.at[j + 1]], add=True)
        return carry

    lax.fori_loop(0, CPT // 2, body, 0)
    gather_wait(CPT, 0)
    plsc.subcore_barrier()
    pltpu.sync_copy(acc.at[pl.ds(s * STRIPE, STRIPE)],
                    out_hbm.at[c, pl.ds(s * STRIPE, STRIPE)])


@functools.cache
def _sc_kernels():
    mesh = plsc.VectorSubcoreMesh(core_axis_name="c", subcore_axis_name="s",
                                  num_cores=NC, num_subcores=NS)
    params = pltpu.CompilerParams(use_tc_tiling_on_sc=False,
                                  needs_layout_passes=False)
    stage1_kernel = pl.kernel(
        _stage1_body,
        out_type=(jax.ShapeDtypeStruct((NC, NACC, D_HID), jnp.float32),
                  jax.ShapeDtypeStruct((NC * NACC, D_HID), jnp.float32),
                  jax.ShapeDtypeStruct((NC, NACC), jnp.float32),
                  jax.ShapeDtypeStruct((E_ROWS, CHUNK), jnp.int32)),
        mesh=mesh,
        scratch_types=[
            pltpu.VMEM((NACC // L, L), jnp.float32),
            pltpu.VMEM((DROWS, CHUNK), jnp.int32),
            pltpu.VMEM((DROWS, CHUNK), jnp.int32),
            pltpu.VMEM((NS, NVEC, L), jnp.float32),
            pltpu.VMEM((STRIPE,), jnp.float32),
            pltpu.VMEM((STRIPE, D_HID), jnp.float32),
            pltpu.VMEM((CPT + 8, CHUNK), jnp.int32),
            pltpu.VMEM((CPT, CHUNK), jnp.int32),
            pltpu.VMEM((2, CHUNK, D_HID), jnp.float32),
            pltpu.VMEM_SHARED((NS, NACC // L, L), jnp.float32),
            pltpu.VMEM_SHARED((NACC, D_HID), jnp.float32),
            pltpu.SemaphoreType.DMA,
            pltpu.SemaphoreType.DMA,
        ],
        compiler_params=params,
    )
    prop_kernel = pl.kernel(
        _prop_body,
        out_type=jax.ShapeDtypeStruct((NC, NACC, D_HID), jnp.float32),
        mesh=mesh,
        scratch_types=[
            pltpu.VMEM((CPT + 8, CHUNK), jnp.int32),
            pltpu.VMEM((CPT, CHUNK), jnp.int32),
            pltpu.VMEM((2, CHUNK, D_HID), jnp.float32),
            pltpu.VMEM_SHARED((NACC, D_HID), jnp.float32),
            pltpu.SemaphoreType.DMA,
            pltpu.SemaphoreType.DMA,
        ],
        compiler_params=params,
    )
    return stage1_kernel, prop_kernel


_RB = 1000


def _lin1_body(x_ref, w1_ref, u_ref):
    u_ref[...] = jnp.dot(x_ref[...], w1_ref[...],
                         preferred_element_type=jnp.float32)


def _lin1(x, W1):
    return pl.pallas_call(
        _lin1_body,
        grid=(N // _RB,),
        in_specs=[
            pl.BlockSpec((_RB, D_IN), lambda i: (i, 0)),
            pl.BlockSpec((D_IN, D_HID), lambda i: (0, 0)),
        ],
        out_specs=pl.BlockSpec((_RB, D_HID), lambda i: (i, 0)),
        out_shape=jax.ShapeDtypeStruct((N, D_HID), jnp.float32),
    )(x, W1)


def _hid_body(s1_ref, g1_ref, dinv_ref, b1_ref, g2_ref):
    dinv = dinv_ref[...]
    pre = dinv * (s1_ref[0] + s1_ref[1] + g1_ref[...]) + b1_ref[...]
    g2_ref[...] = dinv * jnp.maximum(pre, 0.0)


def _hid(s1, g1, dinv, b1):
    return pl.pallas_call(
        _hid_body,
        grid=(N // _RB,),
        in_specs=[
            pl.BlockSpec((NC, _RB, D_HID), lambda i: (0, i, 0)),
            pl.BlockSpec((_RB, D_HID), lambda i: (i, 0)),
            pl.BlockSpec((_RB, 1), lambda i: (i, 0)),
            pl.BlockSpec((1, D_HID), lambda i: (0, 0)),
        ],
        out_specs=pl.BlockSpec((_RB, D_HID), lambda i: (i, 0)),
        out_shape=jax.ShapeDtypeStruct((N, D_HID), jnp.float32),
    )(s1, g1, dinv, b1)


def _out_body(s2_ref, g2_ref, dinv_ref, w2_ref, b2_ref, out_ref):
    h = dinv_ref[...] * (s2_ref[0] + s2_ref[1] + g2_ref[...])
    z = jnp.dot(h, w2_ref[...], preferred_element_type=jnp.float32) + b2_ref[...]
    m = jnp.max(z, axis=1, keepdims=True)
    lse = jnp.log(jnp.sum(jnp.exp(z - m), axis=1, keepdims=True)) + m
    out_ref[...] = z - lse


def _out(s2, g2, dinv, W2, b2):
    return pl.pallas_call(
        _out_body,
        grid=(N // _RB,),
        in_specs=[
            pl.BlockSpec((NC, _RB, D_HID), lambda i: (0, i, 0)),
            pl.BlockSpec((_RB, D_HID), lambda i: (i, 0)),
            pl.BlockSpec((_RB, 1), lambda i: (i, 0)),
            pl.BlockSpec((D_HID, D_OUT), lambda i: (0, 0)),
            pl.BlockSpec((1, D_OUT), lambda i: (0, 0)),
        ],
        out_specs=pl.BlockSpec((_RB, D_OUT), lambda i: (i, 0)),
        out_shape=jax.ShapeDtypeStruct((N, D_OUT), jnp.float32),
    )(s2, g2, dinv, W2, b2)


def kernel(x, edge_index, W1, b1, W2, b2):
    src = edge_index[0].astype(jnp.int32)
    dst = edge_index[1].astype(jnp.int32)
    pad = E_PAD - E
    src_p = jnp.concatenate([src, jnp.zeros((pad,), jnp.int32)]).reshape(E_ROWS, CHUNK)
    dst_p = jnp.concatenate([dst, jnp.full((pad,), DUMMY, jnp.int32)]).reshape(E_ROWS, CHUNK)

    zeros_stripe = jnp.zeros((STRIPE, D_HID), jnp.float32)

    stage1_kernel, prop_kernel = _sc_kernels()

    u = _lin1(x, W1)
    u_pad = jnp.pad(u, ((0, NACC - N), (0, 0)))
    s1, g1, dinv2, dstp = stage1_kernel(u_pad, src_p, dst_p, zeros_stripe)
    dinv = dinv2[0, :N].reshape(N, 1)
    g2 = _hid(s1[:, :N], g1[:N], dinv, b1.reshape(1, D_HID))

    g2_pad = jnp.pad(g2, ((0, NACC - N), (0, 0)))
    s2 = prop_kernel(g2_pad, src_p, dstp, zeros_stripe)
    return _out(s2[:, :N], g2, dinv, W2, b2.reshape(1, D_OUT))


# speedup vs baseline: 1.1490x; 1.0209x over previous
import functools

import jax
import jax.numpy as jnp
from jax import lax
from jax.experimental import pallas as pl
from jax.experimental.pallas import tpu as pltpu
from jax.experimental.pallas import tpu_sc as plsc

N = 10000
E = 320000
D_IN = 128
D_HID = 16
D_OUT = 64

NC = 2
NS = 16
NW = NC * NS

CHUNK = 128
CPT = 80
EPT = CPT * CHUNK
E_PAD = EPT * NW
E_ROWS = E_PAD // CHUNK
DROWS = E_ROWS // NS

NACC = 10240
STRIPE = NACC // NS
DUMMY = N

L = 16
NVEC = STRIPE // L


def _take16(y, idx):
    dnums = lax.GatherDimensionNumbers(offset_dims=(), collapsed_slice_dims=(0,),
                                       start_index_map=(0,))
    return lax.gather(y, idx[:, None], dnums, slice_sizes=(1,),
                      mode=lax.GatherScatterMode.PROMISE_IN_BOUNDS)


def _rsqrt16(d):
    i = plsc.bitcast(d, jnp.int32)
    i = jnp.int32(0x5F3759DF) - lax.shift_right_logical(i, 1)
    y = plsc.bitcast(i, jnp.float32)
    for _ in range(3):
        y = y * (1.5 - 0.5 * d * y * y)
    return y


def _stage1_body(u_hbm, src_hbm, dst_hbm, zeros_hbm,
                 s1_hbm, g1_hbm, dinv_hbm, dstp_hbm,
                 hist, sdg, ddg, tmp, dvbuf, g1buf, sps, spd, rows,
                 spm, acc, gs0, gs1):
    c = lax.axis_index("c")
    s = lax.axis_index("s")
    tid = c * NS + s
    zero16 = jnp.zeros((16,), jnp.int32)
    ones16 = jnp.ones((16,), jnp.float32)

    pltpu.sync_copy(zeros_hbm, hist)
    pltpu.sync_copy(src_hbm.at[pl.ds(s * DROWS, DROWS)], sdg)
    pltpu.sync_copy(dst_hbm.at[pl.ds(s * DROWS, DROWS)], ddg)

    def hist_row(j, carry):
        for k in range(CHUNK // L):
            sv = sdg[j, pl.ds(k * L, L)]
            dv = ddg[j, pl.ds(k * L, L)]
            dp = jnp.where(sv == dv, jnp.int32(DUMMY), dv)
            ddg[j, pl.ds(k * L, L)] = dp
            plsc.addupdate_scatter(
                hist, [lax.shift_right_logical(dp, 4), dp & 15], ones16)
        return carry

    lax.fori_loop(0, DROWS, hist_row, 0)

    pltpu.sync_copy(src_hbm.at[pl.ds(tid * CPT, CPT)], sps.at[pl.ds(0, CPT)])
    pltpu.sync_copy(dst_hbm.at[pl.ds(tid * CPT, CPT)], spd)
    slab = c * NACC

    def prep_row(j, carry):
        for k in range(CHUNK // L):
            sv = sps[j, pl.ds(k * L, L)]
            dv = spd[j, pl.ds(k * L, L)]
            spd[j, pl.ds(k * L, L)] = jnp.where(sv == dv, jnp.int32(DUMMY), dv)
            sps[j, pl.ds(k * L, L)] = sv + slab
        return carry

    lax.fori_loop(0, CPT, prep_row, 0)
    for k in range(CHUNK // L):
        sps[CPT, pl.ds(k * L, L)] = zero16 + slab

    pltpu.sync_copy(hist, spm.at[s])
    pltpu.sync_copy(zeros_hbm, acc.at[pl.ds(s * STRIPE, STRIPE)])
    @pl.when(c == 0)
    def _():
        pltpu.sync_copy(ddg, dstp_hbm.at[pl.ds(s * DROWS, DROWS)])
    plsc.subcore_barrier()

    for t in range(NS):
        pltpu.sync_copy(spm.at[t, pl.ds(s * NVEC, NVEC)], tmp.at[t])
    pltpu.sync_copy(u_hbm.at[pl.ds(s * STRIPE, STRIPE)], g1buf)

    lane_ids = [jnp.full((L,), i, jnp.int32) for i in range(L)]

    def dinv_vec(v, carry):
        deg = tmp[0, v, :] + 1.0
        for t in range(1, NS):
            deg = deg + tmp[t, v, :]
        y = _rsqrt16(deg)
        dvbuf[pl.ds(v * L, L)] = y
        for i in range(L):
            row = v * L + i
            g1buf[row, :] = g1buf[row, :] * _take16(y, lane_ids[i])
        return carry

    lax.fori_loop(0, NVEC, dinv_vec, 0)
    pltpu.sync_copy(g1buf, g1_hbm.at[pl.ds(slab + s * STRIPE, STRIPE)])
    pltpu.sync_copy(dvbuf, dinv_hbm.at[c, pl.ds(s * STRIPE, STRIPE)])
    plsc.subcore_barrier()

    sems = (gs0, gs1)

    def gather(j, b):
        return pltpu.async_copy(g1_hbm.at[sps.at[j]], rows.at[b], sems[b])

    def gather_wait(j, b):
        pltpu.make_async_copy(g1_hbm.at[sps.at[j]], rows.at[b], sems[b]).wait()

    gather(0, 0)

    def prop(t, carry):
        j = 2 * t
        gather(j + 1, 1)
        gather_wait(j, 0)
        pltpu.sync_copy(rows.at[0], acc.at[spd.at[j]], add=True)
        gather(j + 2, 0)
        gather_wait(j + 1, 1)
        pltpu.sync_copy(rows.at[1], acc.at[spd.at[j + 1]], add=True)
        return carry

    lax.fori_loop(0, CPT // 2, prop, 0)
    gather_wait(CPT, 0)
    plsc.subcore_barrier()
    pltpu.sync_copy(acc.at[pl.ds(s * STRIPE, STRIPE)],
                    s1_hbm.at[c, pl.ds(s * STRIPE, STRIPE)])


def _prop_body(g_hbm, src_hbm, dstp_hbm, zeros_hbm,
               out_hbm,
               idx_s, idx_d, rows, acc, gs0, gs1):
    c = lax.axis_index("c")
    s = lax.axis_index("s")
    tid = c * NS + s
    base_row = tid * CPT
    pltpu.sync_copy(zeros_hbm, acc.at[pl.ds(s * STRIPE, STRIPE)])
    pltpu.sync_copy(src_hbm.at[pl.ds(base_row, CPT)], idx_s.at[pl.ds(0, CPT)])
    pltpu.sync_copy(dstp_hbm.at[pl.ds(base_row, CPT)], idx_d)
    zero16 = jnp.zeros((16,), jnp.int32)
    for k in range(CHUNK // L):
        idx_s[CPT, pl.ds(k * L, L)] = zero16
    plsc.subcore_barrier()

    sems = (gs0, gs1)

    def gather(j, b):
        return pltpu.async_copy(g_hbm.at[idx_s.at[j]], rows.at[b], sems[b])

    def gather_wait(j, b):
        pltpu.make_async_copy(g_hbm.at[idx_s.at[j]], rows.at[b], sems[b]).wait()

    gather(0, 0)

    def body(t, carry):
        j = 2 * t
        gather(j + 1, 1)
        gather_wait(j, 0)
        pltpu.sync_copy(rows.at[0], acc.at[idx_d.at[j]], add=True)
        gather(j + 2, 0)
        gather_wait(j + 1, 1)
        pltpu.sync_copy(rows.at[1], acc.at[idx_d.at[j + 1]], add=True)
        return carry

    lax.fori_loop(0, CPT // 2, body, 0)
    gather_wait(CPT, 0)
    plsc.subcore_barrier()
    pltpu.sync_copy(acc.at[pl.ds(s * STRIPE, STRIPE)],
                    out_hbm.at[c, pl.ds(s * STRIPE, STRIPE)])


@functools.cache
def _sc_kernels():
    mesh = plsc.VectorSubcoreMesh(core_axis_name="c", subcore_axis_name="s",
                                  num_cores=NC, num_subcores=NS)
    params = pltpu.CompilerParams(use_tc_tiling_on_sc=False,
                                  needs_layout_passes=False)
    stage1_kernel = pl.kernel(
        _stage1_body,
        out_type=(jax.ShapeDtypeStruct((NC, NACC, D_HID), jnp.float32),
                  jax.ShapeDtypeStruct((NC * NACC, D_HID), jnp.float32),
                  jax.ShapeDtypeStruct((NC, NACC), jnp.float32),
                  jax.ShapeDtypeStruct((E_ROWS, CHUNK), jnp.int32)),
        mesh=mesh,
        scratch_types=[
            pltpu.VMEM((NACC // L, L), jnp.float32),
            pltpu.VMEM((DROWS, CHUNK), jnp.int32),
            pltpu.VMEM((DROWS, CHUNK), jnp.int32),
            pltpu.VMEM((NS, NVEC, L), jnp.float32),
            pltpu.VMEM((STRIPE,), jnp.float32),
            pltpu.VMEM((STRIPE, D_HID), jnp.float32),
            pltpu.VMEM((CPT + 8, CHUNK), jnp.int32),
            pltpu.VMEM((CPT, CHUNK), jnp.int32),
            pltpu.VMEM((2, CHUNK, D_HID), jnp.float32),
            pltpu.VMEM_SHARED((NS, NACC // L, L), jnp.float32),
            pltpu.VMEM_SHARED((NACC, D_HID), jnp.float32),
            pltpu.SemaphoreType.DMA,
            pltpu.SemaphoreType.DMA,
        ],
        compiler_params=params,
    )
    prop_kernel = pl.kernel(
        _prop_body,
        out_type=jax.ShapeDtypeStruct((NC, NACC, D_HID), jnp.float32),
        mesh=mesh,
        scratch_types=[
            pltpu.VMEM((CPT + 8, CHUNK), jnp.int32),
            pltpu.VMEM((CPT, CHUNK), jnp.int32),
            pltpu.VMEM((2, CHUNK, D_HID), jnp.float32),
            pltpu.VMEM_SHARED((NACC, D_HID), jnp.float32),
            pltpu.SemaphoreType.DMA,
            pltpu.SemaphoreType.DMA,
        ],
        compiler_params=params,
    )
    return stage1_kernel, prop_kernel


_RB = 10000


def _lin1_body(x_ref, w1_ref, u_ref):
    u_ref[...] = jnp.dot(x_ref[...], w1_ref[...],
                         preferred_element_type=jnp.float32)


def _lin1(x, W1):
    return pl.pallas_call(
        _lin1_body,
        grid=(N // _RB,),
        in_specs=[
            pl.BlockSpec((_RB, D_IN), lambda i: (i, 0)),
            pl.BlockSpec((D_IN, D_HID), lambda i: (0, 0)),
        ],
        out_specs=pl.BlockSpec((_RB, D_HID), lambda i: (i, 0)),
        out_shape=jax.ShapeDtypeStruct((N, D_HID), jnp.float32),
    )(x, W1)


def _hid_body(s1_ref, g1_ref, dinv_ref, b1_ref, g2_ref):
    dinv = dinv_ref[...]
    pre = dinv * (s1_ref[0] + s1_ref[1] + g1_ref[...]) + b1_ref[...]
    g2_ref[...] = dinv * jnp.maximum(pre, 0.0)


def _hid(s1, g1, dinv, b1):
    return pl.pallas_call(
        _hid_body,
        grid=(N // _RB,),
        in_specs=[
            pl.BlockSpec((NC, _RB, D_HID), lambda i: (0, i, 0)),
            pl.BlockSpec((_RB, D_HID), lambda i: (i, 0)),
            pl.BlockSpec((_RB, 1), lambda i: (i, 0)),
            pl.BlockSpec((1, D_HID), lambda i: (0, 0)),
        ],
        out_specs=pl.BlockSpec((_RB, D_HID), lambda i: (i, 0)),
        out_shape=jax.ShapeDtypeStruct((N, D_HID), jnp.float32),
    )(s1, g1, dinv, b1)


def _out_body(s2_ref, g2_ref, dinv_ref, w2_ref, b2_ref, out_ref):
    h = dinv_ref[...] * (s2_ref[0] + s2_ref[1] + g2_ref[...])
    z = jnp.dot(h, w2_ref[...], preferred_element_type=jnp.float32) + b2_ref[...]
    m = jnp.max(z, axis=1, keepdims=True)
    lse = jnp.log(jnp.sum(jnp.exp(z - m), axis=1, keepdims=True)) + m
    out_ref[...] = z - lse


def _out(s2, g2, dinv, W2, b2):
    return pl.pallas_call(
        _out_body,
        grid=(N // _RB,),
        in_specs=[
            pl.BlockSpec((NC, _RB, D_HID), lambda i: (0, i, 0)),
            pl.BlockSpec((_RB, D_HID), lambda i: (i, 0)),
            pl.BlockSpec((_RB, 1), lambda i: (i, 0)),
            pl.BlockSpec((D_HID, D_OUT), lambda i: (0, 0)),
            pl.BlockSpec((1, D_OUT), lambda i: (0, 0)),
        ],
        out_specs=pl.BlockSpec((_RB, D_OUT), lambda i: (i, 0)),
        out_shape=jax.ShapeDtypeStruct((N, D_OUT), jnp.float32),
    )(s2, g2, dinv, W2, b2)


def kernel(x, edge_index, W1, b1, W2, b2):
    src = edge_index[0].astype(jnp.int32)
    dst = edge_index[1].astype(jnp.int32)
    pad = E_PAD - E
    src_p = jnp.concatenate([src, jnp.zeros((pad,), jnp.int32)]).reshape(E_ROWS, CHUNK)
    dst_p = jnp.concatenate([dst, jnp.full((pad,), DUMMY, jnp.int32)]).reshape(E_ROWS, CHUNK)

    zeros_stripe = jnp.zeros((STRIPE, D_HID), jnp.float32)

    stage1_kernel, prop_kernel = _sc_kernels()

    u = _lin1(x, W1)
    u_pad = jnp.pad(u, ((0, NACC - N), (0, 0)))
    s1, g1, dinv2, dstp = stage1_kernel(u_pad, src_p, dst_p, zeros_stripe)
    dinv = dinv2[0, :N].reshape(N, 1)
    g2 = _hid(s1[:, :N], g1[:N], dinv, b1.reshape(1, D_HID))

    g2_pad = jnp.pad(g2, ((0, NACC - N), (0, 0)))
    s2 = prop_kernel(g2_pad, src_p, dstp, zeros_stripe)
    return _out(s2[:, :N], g2, dinv, W2, b2.reshape(1, D_OUT))


# speedup vs baseline: 1.2067x; 1.0502x over previous
import functools

import jax
import jax.numpy as jnp
from jax import lax
from jax.experimental import pallas as pl
from jax.experimental.pallas import tpu as pltpu
from jax.experimental.pallas import tpu_sc as plsc

N = 10000
E = 320000
D_IN = 128
D_HID = 16
D_OUT = 64

NC = 2
NS = 16
NW = NC * NS

CHUNK = 128
CPT = 80
EPT = CPT * CHUNK
E_PAD = EPT * NW
E_ROWS = E_PAD // CHUNK
DROWS = E_ROWS // NS

CPT0 = 96
CPT1 = 2 * CPT - CPT0
CPT_MAX = max(CPT0, CPT1)
E_ROWS_STAGE = E_ROWS + 32

NACC = 10240
STRIPE = NACC // NS
DUMMY = N

L = 16
NVEC = STRIPE // L


def _take16(y, idx):
    dnums = lax.GatherDimensionNumbers(offset_dims=(), collapsed_slice_dims=(0,),
                                       start_index_map=(0,))
    return lax.gather(y, idx[:, None], dnums, slice_sizes=(1,),
                      mode=lax.GatherScatterMode.PROMISE_IN_BOUNDS)


def _rsqrt16(d):
    i = plsc.bitcast(d, jnp.int32)
    i = jnp.int32(0x5F3759DF) - lax.shift_right_logical(i, 1)
    y = plsc.bitcast(i, jnp.float32)
    for _ in range(3):
        y = y * (1.5 - 0.5 * d * y * y)
    return y


def _stage1_body(u_hbm, src_hbm, dst_hbm, zeros_hbm,
                 s1_hbm, g1_hbm, dinv_hbm, dstp_hbm,
                 hist, sdg, ddg, tmp, dvbuf, g1buf, sps, spd, rows,
                 spm, acc, gs0, gs1):
    c = lax.axis_index("c")
    s = lax.axis_index("s")
    tid = c * NS + s
    zero16 = jnp.zeros((16,), jnp.int32)
    ones16 = jnp.ones((16,), jnp.float32)

    pltpu.sync_copy(zeros_hbm, hist)
    pltpu.sync_copy(src_hbm.at[pl.ds(s * DROWS, DROWS)], sdg)
    pltpu.sync_copy(dst_hbm.at[pl.ds(s * DROWS, DROWS)], ddg)

    def hist_row(j, carry):
        for k in range(CHUNK // L):
            sv = sdg[j, pl.ds(k * L, L)]
            dv = ddg[j, pl.ds(k * L, L)]
            dp = jnp.where(sv == dv, jnp.int32(DUMMY), dv)
            ddg[j, pl.ds(k * L, L)] = dp
            plsc.addupdate_scatter(
                hist, [lax.shift_right_logical(dp, 4), dp & 15], ones16)
        return carry

    lax.fori_loop(0, DROWS, hist_row, 0)

    base_row = jnp.where(c == 0, s * CPT0, NS * CPT0 + s * CPT1)
    pltpu.sync_copy(src_hbm.at[pl.ds(base_row, CPT_MAX)],
                    sps.at[pl.ds(0, CPT_MAX)])
    pltpu.sync_copy(dst_hbm.at[pl.ds(base_row, CPT_MAX)], spd)
    slab = c * NACC

    def prep_row(j, carry):
        for k in range(CHUNK // L):
            sv = sps[j, pl.ds(k * L, L)]
            dv = spd[j, pl.ds(k * L, L)]
            spd[j, pl.ds(k * L, L)] = jnp.where(sv == dv, jnp.int32(DUMMY), dv)
            sps[j, pl.ds(k * L, L)] = sv + slab
        return carry

    lax.fori_loop(0, CPT_MAX, prep_row, 0)

    pltpu.sync_copy(hist, spm.at[s])
    pltpu.sync_copy(zeros_hbm, acc.at[pl.ds(s * STRIPE, STRIPE)])
    @pl.when(c == 0)
    def _():
        pltpu.sync_copy(ddg, dstp_hbm.at[pl.ds(s * DROWS, DROWS)])
    plsc.subcore_barrier()

    for t in range(NS):
        pltpu.sync_copy(spm.at[t, pl.ds(s * NVEC, NVEC)], tmp.at[t])
    pltpu.sync_copy(u_hbm.at[pl.ds(s * STRIPE, STRIPE)], g1buf)

    lane_ids = [jnp.full((L,), i, jnp.int32) for i in range(L)]

    def dinv_vec(v, carry):
        deg = tmp[0, v, :] + 1.0
        for t in range(1, NS):
            deg = deg + tmp[t, v, :]
        y = _rsqrt16(deg)
        dvbuf[pl.ds(v * L, L)] = y
        for i in range(L):
            row = v * L + i
            g1buf[row, :] = g1buf[row, :] * _take16(y, lane_ids[i])
        return carry

    lax.fori_loop(0, NVEC, dinv_vec, 0)
    pltpu.sync_copy(g1buf, g1_hbm.at[pl.ds(slab + s * STRIPE, STRIPE)])
    pltpu.sync_copy(dvbuf, dinv_hbm.at[c, pl.ds(s * STRIPE, STRIPE)])
    plsc.subcore_barrier()

    sems = (gs0, gs1)

    def gather(j, b):
        return pltpu.async_copy(g1_hbm.at[sps.at[j]], rows.at[b], sems[b])

    def gather_wait(j, b):
        pltpu.make_async_copy(g1_hbm.at[sps.at[j]], rows.at[b], sems[b]).wait()

    def prop(t, carry):
        j = 2 * t
        gather(j + 1, 1)
        gather_wait(j, 0)
        pltpu.sync_copy(rows.at[0], acc.at[spd.at[j]], add=True)
        gather(j + 2, 0)
        gather_wait(j + 1, 1)
        pltpu.sync_copy(rows.at[1], acc.at[spd.at[j + 1]], add=True)
        return carry

    def run_prop(cpt):
        for k in range(CHUNK // L):
            sps[cpt, pl.ds(k * L, L)] = zero16
        gather(0, 0)
        lax.fori_loop(0, cpt // 2, prop, 0)
        gather_wait(cpt, 0)

    @pl.when(c == 0)
    def _():
        run_prop(CPT0)

    @pl.when(c == 1)
    def _():
        run_prop(CPT1)

    plsc.subcore_barrier()
    pltpu.sync_copy(acc.at[pl.ds(s * STRIPE, STRIPE)],
                    s1_hbm.at[c, pl.ds(s * STRIPE, STRIPE)])


def _prop_body(g_hbm, src_hbm, dstp_hbm, zeros_hbm,
               out_hbm,
               idx_s, idx_d, rows, acc, gs0, gs1):
    c = lax.axis_index("c")
    s = lax.axis_index("s")
    base_row = jnp.where(c == 0, s * CPT0, NS * CPT0 + s * CPT1)
    pltpu.sync_copy(zeros_hbm, acc.at[pl.ds(s * STRIPE, STRIPE)])
    pltpu.sync_copy(src_hbm.at[pl.ds(base_row, CPT_MAX)],
                    idx_s.at[pl.ds(0, CPT_MAX)])
    pltpu.sync_copy(dstp_hbm.at[pl.ds(base_row, CPT_MAX)], idx_d)
    plsc.subcore_barrier()

    sems = (gs0, gs1)

    def gather(j, b):
        return pltpu.async_copy(g_hbm.at[idx_s.at[j]], rows.at[b], sems[b])

    def gather_wait(j, b):
        pltpu.make_async_copy(g_hbm.at[idx_s.at[j]], rows.at[b], sems[b]).wait()

    def body(t, carry):
        j = 2 * t
        gather(j + 1, 1)
        gather_wait(j, 0)
        pltpu.sync_copy(rows.at[0], acc.at[idx_d.at[j]], add=True)
        gather(j + 2, 0)
        gather_wait(j + 1, 1)
        pltpu.sync_copy(rows.at[1], acc.at[idx_d.at[j + 1]], add=True)
        return carry

    zero16 = jnp.zeros((16,), jnp.int32)

    def run_prop(cpt):
        for k in range(CHUNK // L):
            idx_s[cpt, pl.ds(k * L, L)] = zero16
        gather(0, 0)
        lax.fori_loop(0, cpt // 2, body, 0)
        gather_wait(cpt, 0)

    @pl.when(c == 0)
    def _():
        run_prop(CPT0)

    @pl.when(c == 1)
    def _():
        run_prop(CPT1)

    plsc.subcore_barrier()
    pltpu.sync_copy(acc.at[pl.ds(s * STRIPE, STRIPE)],
                    out_hbm.at[c, pl.ds(s * STRIPE, STRIPE)])


@functools.cache
def _sc_kernels():
    mesh = plsc.VectorSubcoreMesh(core_axis_name="c", subcore_axis_name="s",
                                  num_cores=NC, num_subcores=NS)
    params = pltpu.CompilerParams(use_tc_tiling_on_sc=False,
                                  needs_layout_passes=False)
    stage1_kernel = pl.kernel(
        _stage1_body,
        out_type=(jax.ShapeDtypeStruct((NC, NACC, D_HID), jnp.float32),
                  jax.ShapeDtypeStruct((NC * NACC, D_HID), jnp.float32),
                  jax.ShapeDtypeStruct((NC, NACC), jnp.float32),
                  jax.ShapeDtypeStruct((E_ROWS_STAGE, CHUNK), jnp.int32)),
        mesh=mesh,
        scratch_types=[
            pltpu.VMEM((NACC // L, L), jnp.float32),
            pltpu.VMEM((DROWS, CHUNK), jnp.int32),
            pltpu.VMEM((DROWS, CHUNK), jnp.int32),
            pltpu.VMEM((NS, NVEC, L), jnp.float32),
            pltpu.VMEM((STRIPE,), jnp.float32),
            pltpu.VMEM((STRIPE, D_HID), jnp.float32),
            pltpu.VMEM((CPT_MAX + 8, CHUNK), jnp.int32),
            pltpu.VMEM((CPT_MAX, CHUNK), jnp.int32),
            pltpu.VMEM((2, CHUNK, D_HID), jnp.float32),
            pltpu.VMEM_SHARED((NS, NACC // L, L), jnp.float32),
            pltpu.VMEM_SHARED((NACC, D_HID), jnp.float32),
            pltpu.SemaphoreType.DMA,
            pltpu.SemaphoreType.DMA,
        ],
        compiler_params=params,
    )
    prop_kernel = pl.kernel(
        _prop_body,
        out_type=jax.ShapeDtypeStruct((NC, NACC, D_HID), jnp.float32),
        mesh=mesh,
        scratch_types=[
            pltpu.VMEM((CPT_MAX + 8, CHUNK), jnp.int32),
            pltpu.VMEM((CPT_MAX, CHUNK), jnp.int32),
            pltpu.VMEM((2, CHUNK, D_HID), jnp.float32),
            pltpu.VMEM_SHARED((NACC, D_HID), jnp.float32),
            pltpu.SemaphoreType.DMA,
            pltpu.SemaphoreType.DMA,
        ],
        compiler_params=params,
    )
    return stage1_kernel, prop_kernel


_RB = 10000


def _lin1_body(x_ref, w1_ref, u_ref):
    u_ref[...] = jnp.dot(x_ref[...], w1_ref[...],
                         preferred_element_type=jnp.float32)


def _lin1(x, W1):
    return pl.pallas_call(
        _lin1_body,
        grid=(N // _RB,),
        in_specs=[
            pl.BlockSpec((_RB, D_IN), lambda i: (i, 0)),
            pl.BlockSpec((D_IN, D_HID), lambda i: (0, 0)),
        ],
        out_specs=pl.BlockSpec((_RB, D_HID), lambda i: (i, 0)),
        out_shape=jax.ShapeDtypeStruct((N, D_HID), jnp.float32),
    )(x, W1)


def _hid_body(s1_ref, g1_ref, dinv_ref, b1_ref, g2_ref):
    dinv = dinv_ref[...]
    pre = dinv * (s1_ref[0] + s1_ref[1] + g1_ref[...]) + b1_ref[...]
    g2_ref[...] = dinv * jnp.maximum(pre, 0.0)


def _hid(s1, g1, dinv, b1):
    return pl.pallas_call(
        _hid_body,
        grid=(N // _RB,),
        in_specs=[
            pl.BlockSpec((NC, _RB, D_HID), lambda i: (0, i, 0)),
            pl.BlockSpec((_RB, D_HID), lambda i: (i, 0)),
            pl.BlockSpec((_RB, 1), lambda i: (i, 0)),
            pl.BlockSpec((1, D_HID), lambda i: (0, 0)),
        ],
        out_specs=pl.BlockSpec((_RB, D_HID), lambda i: (i, 0)),
        out_shape=jax.ShapeDtypeStruct((N, D_HID), jnp.float32),
    )(s1, g1, dinv, b1)


def _out_body(s2_ref, g2_ref, dinv_ref, w2_ref, b2_ref, out_ref):
    h = dinv_ref[...] * (s2_ref[0] + s2_ref[1] + g2_ref[...])
    z = jnp.dot(h, w2_ref[...], preferred_element_type=jnp.float32) + b2_ref[...]
    m = jnp.max(z, axis=1, keepdims=True)
    lse = jnp.log(jnp.sum(jnp.exp(z - m), axis=1, keepdims=True)) + m
    out_ref[...] = z - lse


def _out(s2, g2, dinv, W2, b2):
    return pl.pallas_call(
        _out_body,
        grid=(N // _RB,),
        in_specs=[
            pl.BlockSpec((NC, _RB, D_HID), lambda i: (0, i, 0)),
            pl.BlockSpec((_RB, D_HID), lambda i: (i, 0)),
            pl.BlockSpec((_RB, 1), lambda i: (i, 0)),
            pl.BlockSpec((D_HID, D_OUT), lambda i: (0, 0)),
            pl.BlockSpec((1, D_OUT), lambda i: (0, 0)),
        ],
        out_specs=pl.BlockSpec((_RB, D_OUT), lambda i: (i, 0)),
        out_shape=jax.ShapeDtypeStruct((N, D_OUT), jnp.float32),
    )(s2, g2, dinv, W2, b2)


def kernel(x, edge_index, W1, b1, W2, b2):
    src = edge_index[0].astype(jnp.int32)
    dst = edge_index[1].astype(jnp.int32)
    pad = E_ROWS_STAGE * CHUNK - E
    src_p = jnp.concatenate([src, jnp.zeros((pad,), jnp.int32)]).reshape(E_ROWS_STAGE, CHUNK)
    dst_p = jnp.concatenate([dst, jnp.full((pad,), DUMMY, jnp.int32)]).reshape(E_ROWS_STAGE, CHUNK)

    zeros_stripe = jnp.zeros((STRIPE, D_HID), jnp.float32)

    stage1_kernel, prop_kernel = _sc_kernels()

    u = _lin1(x, W1)
    u_pad = jnp.pad(u, ((0, NACC - N), (0, 0)))
    s1, g1, dinv2, dstp = stage1_kernel(u_pad, src_p, dst_p, zeros_stripe)
    dinv = dinv2[0, :N].reshape(N, 1)
    g2 = _hid(s1[:, :N], g1[:N], dinv, b1.reshape(1, D_HID))

    g2_pad = jnp.pad(g2, ((0, NACC - N), (0, 0)))
    s2 = prop_kernel(g2_pad, src_p, dstp, zeros_stripe)
    return _out(s2[:, :N], g2, dinv, W2, b2.reshape(1, D_OUT))
